# baseline (speedup 1.0000x reference)
import functools
import math

import jax
import jax.numpy as jnp
from jax import lax
from jax.experimental import pallas as pl
from jax.experimental.pallas import tpu as pltpu
from jax.experimental.pallas import tpu_sc as plsc

D_MODEL = 64
SCALE = math.sqrt(D_MODEL)
NC, NS, L = 2, 16, 16
NW = NC * NS
BLK = 128
NBUF = 5


@functools.cache
def _make_emb(S0, S1, V):
    assert S0 == NW * BLK and S1 % NBUF == 0
    dtiles = D_MODEL // 8
    s0t = S0 // BLK

    mesh = plsc.VectorSubcoreMesh(core_axis_name="c", subcore_axis_name="s")

    @functools.partial(
        pl.kernel,
        out_type=jax.ShapeDtypeStruct((S1 * dtiles * s0t * 8, BLK), jnp.float32),
        mesh=mesh,
        scratch_types=[
            pltpu.VMEM((S1, BLK), jnp.int32),
            pltpu.VMEM((NBUF, BLK, D_MODEL), jnp.float32),
            pltpu.VMEM((NBUF, D_MODEL, BLK), jnp.float32),
            [pltpu.SemaphoreType.DMA] * NBUF,
            [pltpu.SemaphoreType.DMA] * NBUF,
        ],
        compiler_params=pltpu.CompilerParams(
            use_tc_tiling_on_sc=False, needs_layout_passes=False
        ),
    )
    def emb(lut_hbm, idx_hbm, out_hbm, idx_v, buf, bufT, gsems, wsems):
        wid = lax.axis_index("s") * NC + lax.axis_index("c")
        pltpu.sync_copy(idx_hbm.at[:, pl.ds(wid * BLK, BLK)], idx_v)

        def gather_refs(ch, b):
            return lut_hbm.at[idx_v.at[ch]], buf.at[b]

        def write_refs(ch, b, dt):
            src = bufT.at[b].at[pl.ds(dt * 8, 8)]
            dst = out_hbm.at[
                pl.ds(ch * (D_MODEL * s0t) + dt * (8 * s0t) + wid * 8, 8)
            ]
            return src, dst

        for p in range(NBUF - 1):
            src, dst = gather_refs(p, p)
            pltpu.async_copy(src, dst, gsems[p])

        @pl.loop(0, S1, step=NBUF)
        def outer(i):
            for b in range(NBUF):
                ch = i + b
                nxt = ch + NBUF - 1

                @pl.when(nxt < S1)
                def _():
                    nb = (b + NBUF - 1) % NBUF
                    src, dst = gather_refs(nxt, nb)
                    pltpu.async_copy(src, dst, gsems[nb])

                @pl.when(ch >= NBUF)
                def _():
                    for dt in range(dtiles):
                        src, dst = write_refs(ch - NBUF, b, dt)
                        pltpu.make_async_copy(src, dst, wsems[b]).wait()

                src, dst = gather_refs(ch, b)
                pltpu.make_async_copy(src, dst, gsems[b]).wait()

                svs = []
                for g in range(BLK // L):
                    iv = idx_v[ch, pl.ds(g * L, L)]
                    svs.append(
                        jnp.where(iv == 0, jnp.float32(0.0), jnp.float32(SCALE))
                    )

                @pl.loop(0, D_MODEL, unroll=8)
                def trans_d(d):
                    cols = jnp.full((L,), d, jnp.int32)
                    for g in range(BLK // L):
                        rows = g * L + lax.iota(jnp.int32, L)
                        v = plsc.load_gather(buf.at[b], [rows, cols])
                        bufT[b, d, pl.ds(g * L, L)] = v * svs[g]

                for dt in range(dtiles):
                    src, dst = write_refs(ch, b, dt)
                    pltpu.async_copy(src, dst, wsems[b])

        for b in range(NBUF):
            ch = S1 - NBUF + b
            for dt in range(dtiles):
                src, dst = write_refs(ch, b, dt)
                pltpu.make_async_copy(src, dst, wsems[b]).wait()

    return emb


def kernel(x, lut):
    s0, s1 = x.shape
    xt = x.T.astype(jnp.int32)
    outp = _make_emb(s0, s1, lut.shape[0])(lut, xt)
    out5 = outp.reshape(s1, D_MODEL // 8, s0 // BLK, 8, BLK)
    return out5.transpose(2, 4, 0, 1, 3).reshape(s0, s1, D_MODEL)


# speedup vs baseline: 1.3687x; 1.3687x over previous
import functools
import math

import jax
import jax.numpy as jnp
from jax import lax
from jax.experimental import pallas as pl
from jax.experimental.pallas import tpu as pltpu
from jax.experimental.pallas import tpu_sc as plsc

D_MODEL = 64
SCALE = math.sqrt(D_MODEL)
NC, NS, L = 2, 16, 16
NW = NC * NS
BLK = 128
NBUF = 5


@functools.cache
def _make_emb(S0, S1, V):
    assert S0 == NW * BLK and S1 % NBUF == 0
    dtiles = D_MODEL // 8
    s0t = S0 // BLK

    mesh = plsc.VectorSubcoreMesh(core_axis_name="c", subcore_axis_name="s")

    @functools.partial(
        pl.kernel,
        out_type=jax.ShapeDtypeStruct((S1 * dtiles * s0t * 8, BLK), jnp.float32),
        mesh=mesh,
        scratch_types=[
            pltpu.VMEM((S1, BLK), jnp.int32),
            pltpu.VMEM((NBUF, BLK, D_MODEL), jnp.float32),
            pltpu.VMEM((D_MODEL, BLK + 1), jnp.float32),
            pltpu.VMEM((NBUF, D_MODEL, BLK), jnp.float32),
            [pltpu.SemaphoreType.DMA] * NBUF,
            [pltpu.SemaphoreType.DMA] * NBUF,
        ],
        compiler_params=pltpu.CompilerParams(
            use_tc_tiling_on_sc=False, needs_layout_passes=False
        ),
    )
    def emb(lut_hbm, idx_hbm, out_hbm, idx_v, buf, bufP, bufT, gsems, wsems):
        wid = lax.axis_index("s") * NC + lax.axis_index("c")
        pltpu.sync_copy(idx_hbm.at[:, pl.ds(wid * BLK, BLK)], idx_v)

        def gather_refs(ch, b):
            return lut_hbm.at[idx_v.at[ch]], buf.at[b]

        def write_refs(ch, b, dt):
            src = bufT.at[b].at[pl.ds(dt * 8, 8)]
            dst = out_hbm.at[
                pl.ds(ch * (D_MODEL * s0t) + dt * (8 * s0t) + wid * 8, 8)
            ]
            return src, dst

        for p in range(NBUF - 1):
            src, dst = gather_refs(p, p)
            pltpu.async_copy(src, dst, gsems[p])

        @pl.loop(0, S1, step=NBUF)
        def outer(i):
            for b in range(NBUF):
                ch = i + b
                nxt = ch + NBUF - 1

                @pl.when(nxt < S1)
                def _():
                    nb = (b + NBUF - 1) % NBUF
                    src, dst = gather_refs(nxt, nb)
                    pltpu.async_copy(src, dst, gsems[nb])

                @pl.when(ch >= NBUF)
                def _():
                    for dt in range(dtiles):
                        src, dst = write_refs(ch - NBUF, b, dt)
                        pltpu.make_async_copy(src, dst, wsems[b]).wait()

                src, dst = gather_refs(ch, b)
                pltpu.make_async_copy(src, dst, gsems[b]).wait()

                @pl.loop(0, BLK, unroll=8)
                def trans_r(r):
                    colr = jnp.full((L,), r, jnp.int32)
                    for c in range(D_MODEL // L):
                        v = buf[b, r, pl.ds(c * L, L)]
                        rows = c * L + lax.iota(jnp.int32, L)
                        plsc.store_scatter(bufP, [rows, colr], v)

                svs = []
                for g in range(BLK // L):
                    iv = idx_v[ch, pl.ds(g * L, L)]
                    svs.append(
                        jnp.where(iv == 0, jnp.float32(0.0), jnp.float32(SCALE))
                    )

                @pl.loop(0, D_MODEL, unroll=8)
                def scale_d(d):
                    for g in range(BLK // L):
                        sl = bufP[d, pl.ds(g * L, L)]
                        bufT[b, d, pl.ds(g * L, L)] = sl * svs[g]

                for dt in range(dtiles):
                    src, dst = write_refs(ch, b, dt)
                    pltpu.async_copy(src, dst, wsems[b])

        for b in range(NBUF):
            ch = S1 - NBUF + b
            for dt in range(dtiles):
                src, dst = write_refs(ch, b, dt)
                pltpu.make_async_copy(src, dst, wsems[b]).wait()

    return emb


def kernel(x, lut):
    s0, s1 = x.shape
    xt = x.T.astype(jnp.int32)
    outp = _make_emb(s0, s1, lut.shape[0])(lut, xt)
    out5 = outp.reshape(s1, D_MODEL // 8, s0 // BLK, 8, BLK)
    return out5.transpose(2, 4, 0, 1, 3).reshape(s0, s1, D_MODEL)


# speedup vs baseline: 1.3710x; 1.0017x over previous
import functools
import math

import jax
import jax.numpy as jnp
from jax import lax
from jax.experimental import pallas as pl
from jax.experimental.pallas import tpu as pltpu
from jax.experimental.pallas import tpu_sc as plsc

D_MODEL = 64
SCALE = math.sqrt(D_MODEL)
NC, NS, L = 2, 16, 16
NW = NC * NS
BLK = 128
NBUF = 5


@functools.cache
def _make_emb(S0, S1, V):
    assert S0 == NW * BLK and S1 % NBUF == 0
    dtiles = D_MODEL // 8
    s0t = S0 // BLK

    mesh = plsc.VectorSubcoreMesh(core_axis_name="c", subcore_axis_name="s")

    @functools.partial(
        pl.kernel,
        out_type=jax.ShapeDtypeStruct((S1 * dtiles, s0t * 8, BLK), jnp.float32),
        mesh=mesh,
        scratch_types=[
            pltpu.VMEM((S1, BLK), jnp.int32),
            pltpu.VMEM((NBUF, BLK, D_MODEL), jnp.float32),
            pltpu.VMEM((D_MODEL, BLK + 1), jnp.float32),
            pltpu.VMEM((NBUF, dtiles, 8, BLK), jnp.float32),
            [pltpu.SemaphoreType.DMA] * NBUF,
            [pltpu.SemaphoreType.DMA] * NBUF,
        ],
        compiler_params=pltpu.CompilerParams(
            use_tc_tiling_on_sc=False, needs_layout_passes=False
        ),
    )
    def emb(lut_hbm, idx_hbm, out_hbm, idx_v, buf, bufP, bufT, gsems, wsems):
        wid = lax.axis_index("s") * NC + lax.axis_index("c")
        pltpu.sync_copy(idx_hbm.at[:, pl.ds(wid * BLK, BLK)], idx_v)

        def gather_refs(ch, b):
            return lut_hbm.at[idx_v.at[ch]], buf.at[b]

        def write_refs(ch, b):
            src = bufT.at[b]
            dst = out_hbm.at[pl.ds(ch * dtiles, dtiles), pl.ds(wid * 8, 8)]
            return src, dst

        for p in range(NBUF - 1):
            src, dst = gather_refs(p, p)
            pltpu.async_copy(src, dst, gsems[p])

        ci = lax.iota(jnp.int32, L)
        rows_c = [c * L + ci for c in range(D_MODEL // L)]

        @pl.loop(0, S1, step=NBUF)
        def outer(i):
            for b in range(NBUF):
                ch = i + b
                nxt = ch + NBUF - 1

                @pl.when(nxt < S1)
                def _():
                    nb = (b + NBUF - 1) % NBUF
                    src, dst = gather_refs(nxt, nb)
                    pltpu.async_copy(src, dst, gsems[nb])

                @pl.when(ch >= NBUF)
                def _():
                    src, dst = write_refs(ch - NBUF, b)
                    pltpu.make_async_copy(src, dst, wsems[b]).wait()

                src, dst = gather_refs(ch, b)
                pltpu.make_async_copy(src, dst, gsems[b]).wait()

                @pl.loop(0, BLK, unroll=8)
                def trans_r(r):
                    colr = jnp.full((L,), r, jnp.int32)
                    for c in range(D_MODEL // L):
                        v = buf[b, r, pl.ds(c * L, L)]
                        plsc.store_scatter(bufP, [rows_c[c], colr], v)

                svs = []
                for g in range(BLK // L):
                    iv = idx_v[ch, pl.ds(g * L, L)]
                    svs.append(
                        jnp.where(iv == 0, jnp.float32(0.0), jnp.float32(SCALE))
                    )

                @pl.loop(0, dtiles)
                def scale_dt(dt):
                    for dd in range(8):
                        for g in range(BLK // L):
                            sl = bufP[dt * 8 + dd, pl.ds(g * L, L)]
                            bufT[b, dt, dd, pl.ds(g * L, L)] = sl * svs[g]

                src, dst = write_refs(ch, b)
                pltpu.async_copy(src, dst, wsems[b])

        for b in range(NBUF):
            src, dst = write_refs(S1 - NBUF + b, b)
            pltpu.make_async_copy(src, dst, wsems[b]).wait()

    return emb


def kernel(x, lut):
    s0, s1 = x.shape
    xt = x.T.astype(jnp.int32)
    outp = _make_emb(s0, s1, lut.shape[0])(lut, xt)
    out5 = outp.reshape(s1, D_MODEL // 8, s0 // BLK, 8, BLK)
    return out5.transpose(2, 4, 0, 1, 3).reshape(s0, s1, D_MODEL)


# speedup vs baseline: 1.3727x; 1.0012x over previous
import functools
import math

import jax
import jax.numpy as jnp
from jax import lax
from jax.experimental import pallas as pl
from jax.experimental.pallas import tpu as pltpu
from jax.experimental.pallas import tpu_sc as plsc

D_MODEL = 64
SCALE = math.sqrt(D_MODEL)
NC, NS, L = 2, 16, 16
NW = NC * NS
BLK = 128
CS1 = 2
NBUF = 2


@functools.cache
def _make_emb(S0, S1, V):
    assert S0 == NW * BLK and (S1 // CS1) % NBUF == 0
    dtiles = D_MODEL // 8
    s0t = S0 // BLK
    chunks = S1 // CS1
    crows = CS1 * BLK

    mesh = plsc.VectorSubcoreMesh(core_axis_name="c", subcore_axis_name="s")

    @functools.partial(
        pl.kernel,
        out_type=jax.ShapeDtypeStruct((S1 * dtiles, s0t * 8, BLK), jnp.float32),
        mesh=mesh,
        scratch_types=[
            pltpu.VMEM((S1 * BLK,), jnp.int32),
            pltpu.VMEM((NBUF, crows, D_MODEL), jnp.float32),
            pltpu.VMEM((D_MODEL, BLK + 1), jnp.float32),
            pltpu.VMEM((NBUF, CS1 * dtiles, 8, BLK), jnp.float32),
            pltpu.SemaphoreType.DMA,
            [pltpu.SemaphoreType.DMA] * NBUF,
            [pltpu.SemaphoreType.DMA] * NBUF,
        ],
        compiler_params=pltpu.CompilerParams(
            use_tc_tiling_on_sc=False, needs_layout_passes=False
        ),
    )
    def emb(lut_hbm, idx_hbm, out_hbm, idxf, buf, bufP, bufT, isem, gsems, wsems):
        wid = lax.axis_index("s") * NC + lax.axis_index("c")

        def stage_refs(r):
            src = idx_hbm.at[r].at[pl.ds(wid * BLK, BLK)]
            dst = idxf.at[pl.ds(r * BLK, BLK)]
            return src, dst

        @pl.loop(0, S1)
        def stage(r):
            src, dst = stage_refs(r)
            pltpu.async_copy(src, dst, isem)

        @pl.loop(0, S1)
        def stage_wait(r):
            src, dst = stage_refs(r)
            pltpu.make_async_copy(src, dst, isem).wait()

        def gather_refs(ch, b):
            return lut_hbm.at[idxf.at[pl.ds(ch * crows, crows)]], buf.at[b]

        def write_refs(ch, b):
            src = bufT.at[b]
            dst = out_hbm.at[
                pl.ds(ch * CS1 * dtiles, CS1 * dtiles), pl.ds(wid * 8, 8)
            ]
            return src, dst

        for p in range(NBUF - 1):
            src, dst = gather_refs(p, p)
            pltpu.async_copy(src, dst, gsems[p])

        ci = lax.iota(jnp.int32, L)
        rows_c = [c * L + ci for c in range(D_MODEL // L)]

        @pl.loop(0, chunks, step=NBUF)
        def outer(i):
            for b in range(NBUF):
                ch = i + b
                nxt = ch + NBUF - 1

                @pl.when(nxt < chunks)
                def _():
                    nb = (b + NBUF - 1) % NBUF
                    src, dst = gather_refs(nxt, nb)
                    pltpu.async_copy(src, dst, gsems[nb])

                @pl.when(ch >= NBUF)
                def _():
                    src, dst = write_refs(ch - NBUF, b)
                    pltpu.make_async_copy(src, dst, wsems[b]).wait()

                src, dst = gather_refs(ch, b)
                pltpu.make_async_copy(src, dst, gsems[b]).wait()

                for k in range(CS1):
                    @pl.loop(0, BLK, unroll=8)
                    def trans_r(r):
                        colr = jnp.full((L,), r, jnp.int32)
                        for c in range(D_MODEL // L):
                            v = buf[b, k * BLK + r, pl.ds(c * L, L)]
                            plsc.store_scatter(bufP, [rows_c[c], colr], v)

                    svs = []
                    for g in range(BLK // L):
                        iv = idxf[pl.ds(ch * crows + k * BLK + g * L, L)]
                        svs.append(
                            jnp.where(
                                iv == 0, jnp.float32(0.0), jnp.float32(SCALE)
                            )
                        )

                    @pl.loop(0, dtiles)
                    def scale_dt(dt):
                        for dd in range(8):
                            for g in range(BLK // L):
                                sl = bufP[dt * 8 + dd, pl.ds(g * L, L)]
                                bufT[b, k * dtiles + dt, dd, pl.ds(g * L, L)] = (
                                    sl * svs[g]
                                )

                src, dst = write_refs(ch, b)
                pltpu.async_copy(src, dst, wsems[b])

        for b in range(NBUF):
            src, dst = write_refs(chunks - NBUF + b, b)
            pltpu.make_async_copy(src, dst, wsems[b]).wait()

    return emb


def kernel(x, lut):
    s0, s1 = x.shape
    xt = x.T.astype(jnp.int32)
    outp = _make_emb(s0, s1, lut.shape[0])(lut, xt)
    out5 = outp.reshape(s1, D_MODEL // 8, s0 // BLK, 8, BLK)
    return out5.transpose(2, 4, 0, 1, 3).reshape(s0, s1, D_MODEL)


# speedup vs baseline: 1.4344x; 1.0450x over previous
import functools
import math

import jax
import jax.numpy as jnp
from jax import lax
from jax.experimental import pallas as pl
from jax.experimental.pallas import tpu as pltpu
from jax.experimental.pallas import tpu_sc as plsc

D_MODEL = 64
SCALE = math.sqrt(D_MODEL)
NC, NS, L = 2, 16, 16
NW = NC * NS
BLK = 128
CS1 = 1
NBUF = 2


@functools.cache
def _make_emb(S0, S1, V):
    assert S0 == NW * BLK and (S1 // CS1) % NBUF == 0
    dtiles = D_MODEL // 8
    s0t = S0 // BLK
    chunks = S1 // CS1
    crows = CS1 * BLK

    mesh = plsc.VectorSubcoreMesh(core_axis_name="c", subcore_axis_name="s")

    @functools.partial(
        pl.kernel,
        out_type=jax.ShapeDtypeStruct((S1 * dtiles, s0t * 8, BLK), jnp.float32),
        mesh=mesh,
        scratch_types=[
            pltpu.VMEM((S1 * BLK,), jnp.int32),
            pltpu.VMEM((NBUF, crows, 128), jnp.float32),
            pltpu.VMEM((D_MODEL, BLK + 1), jnp.float32),
            pltpu.VMEM((NBUF, CS1 * dtiles, 8, BLK), jnp.float32),
            pltpu.SemaphoreType.DMA,
            [pltpu.SemaphoreType.DMA] * NBUF,
            [pltpu.SemaphoreType.DMA] * NBUF,
        ],
        compiler_params=pltpu.CompilerParams(
            use_tc_tiling_on_sc=False, needs_layout_passes=False
        ),
    )
    def emb(lut_hbm, idx_hbm, out_hbm, idxf, buf, bufP, bufT, isem, gsems, wsems):
        wid = lax.axis_index("s") * NC + lax.axis_index("c")

        def stage_refs(r):
            src = idx_hbm.at[r].at[pl.ds(wid * BLK, BLK)]
            dst = idxf.at[pl.ds(r * BLK, BLK)]
            return src, dst

        @pl.loop(0, S1)
        def stage(r):
            src, dst = stage_refs(r)
            pltpu.async_copy(src, dst, isem)

        @pl.loop(0, S1)
        def stage_wait(r):
            src, dst = stage_refs(r)
            pltpu.make_async_copy(src, dst, isem).wait()

        def gather_refs(ch, b):
            src = lut_hbm.at[idxf.at[pl.ds(ch * crows, crows)]]
            return src, buf.at[b]

        def write_refs(ch, b):
            src = bufT.at[b]
            dst = out_hbm.at[
                pl.ds(ch * CS1 * dtiles, CS1 * dtiles), pl.ds(wid * 8, 8)
            ]
            return src, dst

        for p in range(NBUF - 1):
            src, dst = gather_refs(p, p)
            pltpu.async_copy(src, dst, gsems[p])

        ci = lax.iota(jnp.int32, L)
        rows_c = [c * L + ci for c in range(D_MODEL // L)]

        @pl.loop(0, chunks, step=NBUF)
        def outer(i):
            for b in range(NBUF):
                ch = i + b
                nxt = ch + NBUF - 1

                @pl.when(nxt < chunks)
                def _():
                    nb = (b + NBUF - 1) % NBUF
                    src, dst = gather_refs(nxt, nb)
                    pltpu.async_copy(src, dst, gsems[nb])

                @pl.when(ch >= NBUF)
                def _():
                    src, dst = write_refs(ch - NBUF, b)
                    pltpu.make_async_copy(src, dst, wsems[b]).wait()

                src, dst = gather_refs(ch, b)
                pltpu.make_async_copy(src, dst, gsems[b]).wait()

                for k in range(CS1):
                    @pl.loop(0, BLK, unroll=8)
                    def trans_r(r):
                        colr = jnp.full((L,), r, jnp.int32)
                        for c in range(D_MODEL // L):
                            v = buf[b, k * BLK + r, pl.ds(c * L, L)]
                            plsc.store_scatter(bufP, [rows_c[c], colr], v)

                    svs = []
                    for g in range(BLK // L):
                        iv = idxf[pl.ds(ch * crows + k * BLK + g * L, L)]
                        svs.append(
                            jnp.where(
                                iv == 0, jnp.float32(0.0), jnp.float32(SCALE)
                            )
                        )

                    @pl.loop(0, dtiles)
                    def scale_dt(dt):
                        for dd in range(8):
                            for g in range(BLK // L):
                                sl = bufP[dt * 8 + dd, pl.ds(g * L, L)]
                                bufT[b, k * dtiles + dt, dd, pl.ds(g * L, L)] = (
                                    sl * svs[g]
                                )

                src, dst = write_refs(ch, b)
                pltpu.async_copy(src, dst, wsems[b])

        for b in range(NBUF):
            src, dst = write_refs(chunks - NBUF + b, b)
            pltpu.make_async_copy(src, dst, wsems[b]).wait()

    return emb


def kernel(x, lut):
    s0, s1 = x.shape
    xt = x.T.astype(jnp.int32)
    lutp = jnp.pad(lut, ((0, 0), (0, 128 - D_MODEL)))
    outp = _make_emb(s0, s1, lut.shape[0])(lutp, xt)
    out5 = outp.reshape(s1, D_MODEL // 8, s0 // BLK, 8, BLK)
    return out5.transpose(2, 4, 0, 1, 3).reshape(s0, s1, D_MODEL)


# speedup vs baseline: 2.0281x; 1.4139x over previous
import functools
import math

import jax
import jax.numpy as jnp
from jax import lax
from jax.experimental import pallas as pl
from jax.experimental.pallas import tpu as pltpu
from jax.experimental.pallas import tpu_sc as plsc

D_MODEL = 64
SCALE = math.sqrt(D_MODEL)
NC, NS, L = 2, 16, 16
NW = NC * NS
BLK = 128
CS1 = 1
NBUF = 2


@functools.cache
def _make_emb(S0, S1, V):
    assert S0 == NW * BLK and (S1 // CS1) % NBUF == 0
    dtiles = D_MODEL // 8
    s0t = S0 // BLK
    chunks = S1 // CS1
    crows = CS1 * BLK

    mesh = plsc.VectorSubcoreMesh(core_axis_name="c", subcore_axis_name="s")

    @functools.partial(
        pl.kernel,
        out_type=jax.ShapeDtypeStruct((S1 * dtiles, s0t * 8, BLK), jnp.float32),
        mesh=mesh,
        scratch_types=[
            pltpu.VMEM((S1 * BLK,), jnp.int32),
            pltpu.VMEM((NBUF, crows, 128), jnp.float32),
            pltpu.VMEM((NBUF, dtiles, 8, BLK + 1), jnp.float32),
            pltpu.SemaphoreType.DMA,
            [pltpu.SemaphoreType.DMA] * NBUF,
            [pltpu.SemaphoreType.DMA] * NBUF,
        ],
        compiler_params=pltpu.CompilerParams(
            use_tc_tiling_on_sc=False, needs_layout_passes=False
        ),
    )
    def emb(lut_hbm, idx_hbm, out_hbm, idxf, buf, bufP, isem, gsems, wsems):
        wid = lax.axis_index("s") * NC + lax.axis_index("c")

        def stage_refs(r):
            src = idx_hbm.at[r].at[pl.ds(wid * BLK, BLK)]
            dst = idxf.at[pl.ds(r * BLK, BLK)]
            return src, dst

        @pl.loop(0, S1)
        def stage(r):
            src, dst = stage_refs(r)
            pltpu.async_copy(src, dst, isem)

        @pl.loop(0, S1)
        def stage_wait(r):
            src, dst = stage_refs(r)
            pltpu.make_async_copy(src, dst, isem).wait()

        def gather_refs(ch, b):
            src = lut_hbm.at[idxf.at[pl.ds(ch * crows, crows)]]
            return src, buf.at[b]

        def write_refs(ch, b):
            src = bufP.at[b].at[:, :, pl.ds(0, BLK)]
            dst = out_hbm.at[
                pl.ds(ch * CS1 * dtiles, CS1 * dtiles), pl.ds(wid * 8, 8)
            ]
            return src, dst

        for p in range(NBUF - 1):
            src, dst = gather_refs(p, p)
            pltpu.async_copy(src, dst, gsems[p])

        ci = lax.iota(jnp.int32, L)
        rl = ci % 8
        rh_c = [(c * L + ci) // 8 for c in range(D_MODEL // L)]

        @pl.loop(0, chunks, step=NBUF)
        def outer(i):
            for b in range(NBUF):
                ch = i + b
                nxt = ch + NBUF - 1

                @pl.when(nxt < chunks)
                def _():
                    nb = (b + NBUF - 1) % NBUF
                    src, dst = gather_refs(nxt, nb)
                    pltpu.async_copy(src, dst, gsems[nb])

                @pl.when(ch >= NBUF)
                def _():
                    src, dst = write_refs(ch - NBUF, b)
                    pltpu.make_async_copy(src, dst, wsems[b]).wait()

                src, dst = gather_refs(ch, b)
                pltpu.make_async_copy(src, dst, gsems[b]).wait()

                @pl.loop(0, BLK, unroll=8)
                def trans_r(r):
                    colr = jnp.full((L,), r, jnp.int32)
                    for c in range(D_MODEL // L):
                        v = buf[b, r, pl.ds(c * L, L)]
                        plsc.store_scatter(bufP.at[b], [rh_c[c], rl, colr], v)

                svs = []
                for g in range(BLK // L):
                    iv = idxf[pl.ds(ch * crows + g * L, L)]
                    svs.append(
                        jnp.where(iv == 0, jnp.float32(0.0), jnp.float32(SCALE))
                    )

                @pl.loop(0, dtiles)
                def scale_dt(dt):
                    for dd in range(8):
                        for g in range(BLK // L):
                            sl = bufP[b, dt, dd, pl.ds(g * L, L)]
                            bufP[b, dt, dd, pl.ds(g * L, L)] = sl * svs[g]

                src, dst = write_refs(ch, b)
                pltpu.async_copy(src, dst, wsems[b])

        for b in range(NBUF):
            src, dst = write_refs(chunks - NBUF + b, b)
            pltpu.make_async_copy(src, dst, wsems[b]).wait()

    return emb


def kernel(x, lut):
    s0, s1 = x.shape
    xt = x.T.astype(jnp.int32)
    lutp = jnp.pad(lut, ((0, 0), (0, 128 - D_MODEL)))
    outp = _make_emb(s0, s1, lut.shape[0])(lutp, xt)
    out5 = outp.reshape(s1, D_MODEL // 8, s0 // BLK, 8, BLK)
    return out5.transpose(2, 4, 0, 1, 3).reshape(s0, s1, D_MODEL)
